# Initial kernel scaffold; baseline (speedup 1.0000x reference)
import jax
import jax.numpy as jnp
from jax.experimental import pallas as pl


def kernel(x, I):
    raise NotImplementedError("write your pallas kernel here")


# speedup vs baseline: 8.4563x; 8.4563x over previous
import jax
import jax.numpy as jnp
from jax import lax
from jax.experimental import pallas as pl

DIMK = 64
ROW_BLK = 64


def _onehot_body(x_ref, out_ref):
    ids = x_ref[...]
    k = lax.broadcasted_iota(jnp.int32, (ROW_BLK, 128, DIMK), 2)
    out_ref[...] = (ids[:, :, None] == k).astype(jnp.float32)


def kernel(x, I):
    n = x.shape[0] * x.shape[1]
    xr = x.reshape(n // 128, 128)
    grid = (xr.shape[0] // ROW_BLK,)
    out = pl.pallas_call(
        _onehot_body,
        grid=grid,
        in_specs=[pl.BlockSpec((ROW_BLK, 128), lambda i: (i, 0))],
        out_specs=pl.BlockSpec((ROW_BLK, 128, DIMK), lambda i: (i, 0, 0)),
        out_shape=jax.ShapeDtypeStruct((xr.shape[0], 128, DIMK), jnp.float32),
    )(xr)
    return out.reshape(x.shape[0], x.shape[1], DIMK)
